# baseline (speedup 1.0000x reference)
import jax
import jax.numpy as jnp
from jax import lax
from jax.experimental import pallas as pl
from jax.experimental.pallas import tpu as pltpu
from jax.experimental.pallas import tpu_sc as plsc

LABELS = 1000
FEATURES = 512
BATCH = 16384
LAMBDA1 = 0.01
BN = 2048

NC = 2
NS = 16
NW = NC * NS
ROWS_W = BATCH // NW
CROWS = 64
NCHUNK = ROWS_W // CROWS


def _tc_body(pt_ref, lab_ref, cc_ref, out_ref, acc_ref):
    i = pl.program_id(0)
    p = pt_ref[...]
    m = jnp.max(p, axis=0, keepdims=True)
    s = jnp.sum(jnp.exp(p - m), axis=0, keepdims=True)
    lse_sum = jnp.sum(m + jnp.log(s))

    lab = lab_ref[0]
    row = jax.lax.broadcasted_iota(jnp.int32, (LABELS, BN), 0)
    mask = row == lab
    picked_sum = jnp.sum(jnp.where(mask, p, 0.0))
    counts = jnp.sum(mask.astype(jnp.float32), axis=1, keepdims=True)
    rs = jnp.sum(cc_ref[...], axis=1, keepdims=True)
    rs_sum = jnp.sum(counts * rs)

    part_a = lse_sum - picked_sum

    @pl.when(i == 0)
    def _init():
        acc_ref[0] = part_a
        acc_ref[1] = rs_sum

    @pl.when(i > 0)
    def _acc():
        acc_ref[0] += part_a
        acc_ref[1] += rs_sum

    @pl.when(i == pl.num_programs(0) - 1)
    def _fin():
        out_ref[0, 0] = acc_ref[0]
        out_ref[0, 1] = acc_ref[1]


def _sc_xsum_body(x_hbm, out_hbm, buf0, buf1, accbuf, sem0, sem1):
    wid = lax.axis_index("s") * NC + lax.axis_index("c")
    base = wid * ROWS_W
    bufs = (buf0, buf1)
    sems = (sem0, sem1)

    copies = [None] * NCHUNK
    copies[0] = pltpu.make_async_copy(x_hbm.at[pl.ds(base, CROWS)], bufs[0], sems[0])
    copies[0].start()

    zeros = jnp.zeros((16,), jnp.float32)
    accs = (zeros, zeros, zeros, zeros)
    for g in range(NCHUNK):
        if g + 1 < NCHUNK:
            copies[g + 1] = pltpu.make_async_copy(
                x_hbm.at[pl.ds(base + (g + 1) * CROWS, CROWS)],
                bufs[(g + 1) % 2], sems[(g + 1) % 2])
            copies[g + 1].start()
        copies[g].wait()
        buf = bufs[g % 2]

        def body(k, accs):
            a0, a1, a2, a3 = accs
            for j in range(FEATURES // 64):
                a0 += buf[k, pl.ds(64 * j, 16)]
                a1 += buf[k, pl.ds(64 * j + 16, 16)]
                a2 += buf[k, pl.ds(64 * j + 32, 16)]
                a3 += buf[k, pl.ds(64 * j + 48, 16)]
            return (a0, a1, a2, a3)

        accs = lax.fori_loop(0, CROWS, body, accs)

    accbuf[...] = accs[0] + accs[1] + accs[2] + accs[3]
    pltpu.sync_copy(accbuf, out_hbm.at[wid])


_sc_xsum = pl.kernel(
    _sc_xsum_body,
    out_type=jax.ShapeDtypeStruct((NW, 16), jnp.float32),
    mesh=plsc.VectorSubcoreMesh(core_axis_name="c", subcore_axis_name="s"),
    scratch_types=[
        pltpu.VMEM((CROWS, FEATURES), jnp.float32),
        pltpu.VMEM((CROWS, FEATURES), jnp.float32),
        pltpu.VMEM((16,), jnp.float32),
        pltpu.SemaphoreType.DMA,
        pltpu.SemaphoreType.DMA,
    ],
)


@jax.jit
def kernel(predictions, x, labels, classCenter):
    grid = BATCH // BN
    lab3 = labels.astype(jnp.int32).reshape(grid, 1, BN)

    xs_partials = _sc_xsum(x)

    tc_out = pl.pallas_call(
        _tc_body,
        grid=(grid,),
        in_specs=[
            pl.BlockSpec((LABELS, BN), lambda i: (0, i)),
            pl.BlockSpec((1, 1, BN), lambda i: (i, 0, 0)),
            pl.BlockSpec((LABELS, FEATURES), lambda i: (0, 0)),
        ],
        out_specs=pl.BlockSpec((1, 2), lambda i: (0, 0), memory_space=pltpu.SMEM),
        out_shape=jax.ShapeDtypeStruct((1, 2), jnp.float32),
        scratch_shapes=[pltpu.SMEM((2,), jnp.float32)],
        compiler_params=pltpu.CompilerParams(
            dimension_semantics=("arbitrary",),
            vmem_limit_bytes=100 * 1024 * 1024,
        ),
    )(predictions.T, lab3, classCenter)

    xs = jnp.sum(xs_partials)
    part_b = xs - tc_out[0, 1]
    return (tc_out[0, 0] + (LAMBDA1 / 2.0) * part_b * part_b).reshape(())


# speedup vs baseline: 1.0230x; 1.0230x over previous
import jax
import jax.numpy as jnp
from jax import lax
from jax.experimental import pallas as pl
from jax.experimental.pallas import tpu as pltpu
from jax.experimental.pallas import tpu_sc as plsc

LABELS = 1000
FEATURES = 512
BATCH = 16384
LAMBDA1 = 0.01
BN = 2048

NC = 2
NS = 16
NW = NC * NS
ROWS_W = BATCH // NW
CROWS = 64
NCHUNK = ROWS_W // CROWS


def _tc_body(pt_ref, lab_ref, cc_ref, out_ref, acc_ref):
    i = pl.program_id(0)
    p = pt_ref[...]
    m = jnp.max(p, axis=0, keepdims=True)
    s = jnp.sum(jnp.exp(p - m), axis=0, keepdims=True)
    lse_sum = jnp.sum(m + jnp.log(s))

    lab = lab_ref[0]
    row = jax.lax.broadcasted_iota(jnp.int32, (LABELS, BN), 0)
    mask = row == lab
    picked_sum = jnp.sum(jnp.where(mask, p, 0.0))
    counts = jnp.sum(mask.astype(jnp.float32), axis=1, keepdims=True)
    rs = jnp.sum(cc_ref[...], axis=1, keepdims=True)
    rs_sum = jnp.sum(counts * rs)

    part_a = lse_sum - picked_sum

    @pl.when(i == 0)
    def _init():
        acc_ref[0] = part_a
        acc_ref[1] = rs_sum

    @pl.when(i > 0)
    def _acc():
        acc_ref[0] += part_a
        acc_ref[1] += rs_sum

    @pl.when(i == pl.num_programs(0) - 1)
    def _fin():
        out_ref[0, 0] = acc_ref[0]
        out_ref[0, 1] = acc_ref[1]


def _sc_xsum_body(x_hbm, out_hbm, buf0, buf1, accbuf, sem0, sem1):
    wid = lax.axis_index("s") * NC + lax.axis_index("c")
    base = wid * ROWS_W
    bufs = (buf0, buf1)
    sems = (sem0, sem1)

    copies = [None] * NCHUNK
    copies[0] = pltpu.make_async_copy(x_hbm.at[pl.ds(base, CROWS)], bufs[0], sems[0])
    copies[0].start()

    zeros = jnp.zeros((16,), jnp.float32)
    accs = (zeros, zeros, zeros, zeros)
    for g in range(NCHUNK):
        if g + 1 < NCHUNK:
            copies[g + 1] = pltpu.make_async_copy(
                x_hbm.at[pl.ds(base + (g + 1) * CROWS, CROWS)],
                bufs[(g + 1) % 2], sems[(g + 1) % 2])
            copies[g + 1].start()
        copies[g].wait()
        buf = bufs[g % 2]

        def body(k, accs):
            a0, a1, a2, a3 = accs
            for j in range(FEATURES // 64):
                a0 += buf[k, pl.ds(64 * j, 16)]
                a1 += buf[k, pl.ds(64 * j + 16, 16)]
                a2 += buf[k, pl.ds(64 * j + 32, 16)]
                a3 += buf[k, pl.ds(64 * j + 48, 16)]
            return (a0, a1, a2, a3)

        accs = lax.fori_loop(0, CROWS, body, accs)

    accbuf[...] = accs[0] + accs[1] + accs[2] + accs[3]
    pltpu.sync_copy(accbuf, out_hbm.at[wid])


_sc_xsum = pl.kernel(
    _sc_xsum_body,
    out_type=jax.ShapeDtypeStruct((NW, 16), jnp.float32),
    mesh=plsc.VectorSubcoreMesh(core_axis_name="c", subcore_axis_name="s"),
    scratch_types=[
        pltpu.VMEM((CROWS, FEATURES), jnp.float32),
        pltpu.VMEM((CROWS, FEATURES), jnp.float32),
        pltpu.VMEM((16,), jnp.float32),
        pltpu.SemaphoreType.DMA,
        pltpu.SemaphoreType.DMA,
    ],
)


@jax.jit
def kernel(predictions, x, labels, classCenter):
    grid = BATCH // BN
    lab3 = labels.astype(jnp.int32).reshape(grid, 1, BN)

    xs_partials = _sc_xsum(x)

    tc_out = pl.pallas_call(
        _tc_body,
        grid=(grid,),
        in_specs=[
            pl.BlockSpec((LABELS, BN), lambda i: (0, i)),
            pl.BlockSpec((1, 1, BN), lambda i: (i, 0, 0)),
            pl.BlockSpec((LABELS, FEATURES), lambda i: (0, 0)),
        ],
        out_specs=pl.BlockSpec((1, 2), lambda i: (0, 0), memory_space=pltpu.SMEM),
        out_shape=jax.ShapeDtypeStruct((1, 2), jnp.float32),
        scratch_shapes=[pltpu.SMEM((2,), jnp.float32)],
        compiler_params=pltpu.CompilerParams(
            dimension_semantics=("arbitrary",),
        ),
    )(predictions.T, lab3, classCenter)

    xs = jnp.sum(xs_partials)
    part_b = xs - tc_out[0, 1]
    return (tc_out[0, 0] + (LAMBDA1 / 2.0) * part_b * part_b).reshape(())
